# baseline (speedup 1.0000x reference)
import jax
import jax.numpy as jnp
from jax.experimental import pallas as pl

NT = 512
NS = 256
FEA = 512
GATE = 128
BT = 256


def _msdn_kernel(tf_ref, sf_ref, sel_ref, w_ref, b_ref, out_ref):
    tfb = jnp.maximum(tf_ref[...], 0.0)
    sf = sf_ref[...]
    sfr = jnp.maximum(sf, 0.0)
    w = w_ref[...]
    a = 0.5 * jax.lax.dot_general(
        tfb, w[:, :FEA], (((1,), (1,)), ((), ())),
        preferred_element_type=jnp.float32)
    bmT = 0.5 * (jax.lax.dot_general(
        w[:, FEA:], sfr, (((1,), (1,)), ((), ())),
        preferred_element_type=jnp.float32) + b_ref[...].reshape(GATE, 1))
    a16 = a.astype(jnp.bfloat16)
    bmT16 = bmT.astype(jnp.bfloat16)
    h = jnp.tanh(a16[:, :, None] + bmT16[None, :, :])
    t = h
    n = GATE
    while n > 1:
        n //= 2
        t = t[:, :n, :] + t[:, n:, :]
    s_raw = t[:, 0, :].astype(jnp.float32)
    mask = (sel_ref[...] > 0.0).astype(jnp.float32)
    counts = jnp.sum(mask, axis=1, keepdims=True)
    seg = (0.5 * jnp.dot(mask, sf, preferred_element_type=jnp.float32)
           + (0.5 / GATE) * jnp.dot(mask * s_raw, sf,
                                    preferred_element_type=jnp.float32))
    out_ref[...] = seg / jnp.maximum(counts, 1.0)


@jax.jit
def kernel(target_features, source_features, select_mat, W, b):
    b2 = b.reshape(1, GATE)
    grid = NT // BT
    return pl.pallas_call(
        _msdn_kernel,
        grid=(grid,),
        in_specs=[
            pl.BlockSpec((BT, FEA), lambda i: (i, 0)),
            pl.BlockSpec((NS, FEA), lambda i: (0, 0)),
            pl.BlockSpec((BT, NS), lambda i: (i, 0)),
            pl.BlockSpec((GATE, 2 * FEA), lambda i: (0, 0)),
            pl.BlockSpec((1, GATE), lambda i: (0, 0)),
        ],
        out_specs=pl.BlockSpec((BT, FEA), lambda i: (i, 0)),
        out_shape=jax.ShapeDtypeStruct((NT, FEA), jnp.float32),
    )(target_features, source_features, select_mat, W, b2)


# speedup vs baseline: 1.0262x; 1.0262x over previous
import jax
import jax.numpy as jnp
from jax.experimental import pallas as pl

NT = 512
NS = 256
FEA = 512
GATE = 128
BT = 512


def _msdn_kernel(tf_ref, sf_ref, sel_ref, w_ref, b_ref, out_ref):
    tfb = jnp.maximum(tf_ref[...], 0.0)
    sf = sf_ref[...]
    sfr = jnp.maximum(sf, 0.0)
    w = w_ref[...]
    a = 0.5 * jax.lax.dot_general(
        tfb, w[:, :FEA], (((1,), (1,)), ((), ())),
        preferred_element_type=jnp.float32)
    bmT = 0.5 * (jax.lax.dot_general(
        w[:, FEA:], sfr, (((1,), (1,)), ((), ())),
        preferred_element_type=jnp.float32) + b_ref[...].reshape(GATE, 1))
    a16 = a.astype(jnp.bfloat16)
    bmT16 = bmT.astype(jnp.bfloat16)
    h = jnp.tanh(a16[:, :, None] + bmT16[None, :, :])
    acc = h
    n = GATE
    while n > 1:
        n //= 2
        acc = acc[:, :n, :] + acc[:, n:, :]
    s_raw = acc[:, 0, :].astype(jnp.float32)
    mask = (sel_ref[...] > 0.0).astype(jnp.float32)
    counts = jnp.sum(mask, axis=1, keepdims=True)
    seg = (0.5 * jnp.dot(mask, sf, preferred_element_type=jnp.float32)
           + (0.5 / GATE) * jnp.dot(mask * s_raw, sf,
                                    preferred_element_type=jnp.float32))
    out_ref[...] = seg / jnp.maximum(counts, 1.0)


@jax.jit
def kernel(target_features, source_features, select_mat, W, b):
    b2 = b.reshape(1, GATE)
    grid = NT // BT
    return pl.pallas_call(
        _msdn_kernel,
        grid=(grid,),
        in_specs=[
            pl.BlockSpec((BT, FEA), lambda i: (i, 0)),
            pl.BlockSpec((NS, FEA), lambda i: (0, 0)),
            pl.BlockSpec((BT, NS), lambda i: (i, 0)),
            pl.BlockSpec((GATE, 2 * FEA), lambda i: (0, 0)),
            pl.BlockSpec((1, GATE), lambda i: (0, 0)),
        ],
        out_specs=pl.BlockSpec((BT, FEA), lambda i: (i, 0)),
        out_shape=jax.ShapeDtypeStruct((NT, FEA), jnp.float32),
    )(target_features, source_features, select_mat, W, b2)
